# baseline (speedup 1.0000x reference)
import jax
import jax.numpy as jnp
from jax import lax
from jax.experimental import pallas as pl

B, C, H, W = 64, 768, 32, 32
HW = H * W
HALF = C // 2

B_BLK = 8
C_BLK = 128
N_ROW_BLKS = HALF // C_BLK


def _body(emb_ref, fm_ref, out_ref):
    j = pl.program_id(1)
    e = emb_ref[...]
    fm = fm_ref[...]
    ii = lax.broadcasted_iota(jnp.int32, (H, HW), 0)
    jj = lax.broadcasted_iota(jnp.int32, (H, HW), 1)
    sel_row = ((jj // W) == ii).astype(jnp.float32)
    sel_col = ((jj % W) == ii).astype(jnp.float32)
    sel = jnp.where(j < N_ROW_BLKS, sel_row, sel_col)
    pos = lax.dot(e, sel, preferred_element_type=jnp.float32)
    out_ref[...] = fm + pos[None]


def kernel(feature_map, row_embed, col_embed):
    emb = jnp.concatenate([row_embed.T, col_embed.T], axis=0)
    fm3 = feature_map.reshape(B, C, HW)

    grid = (B // B_BLK, C // C_BLK)
    out = pl.pallas_call(
        _body,
        grid=grid,
        in_specs=[
            pl.BlockSpec((C_BLK, H), lambda i, j: (j, 0)),
            pl.BlockSpec((B_BLK, C_BLK, HW), lambda i, j: (i, j, 0)),
        ],
        out_specs=pl.BlockSpec((B_BLK, C_BLK, HW), lambda i, j: (i, j, 0)),
        out_shape=jax.ShapeDtypeStruct((B, C, HW), jnp.float32),
    )(emb, fm3)
    return out.reshape(B, C, H, W)
